# baseline (speedup 1.0000x reference)
import functools

import jax
import jax.numpy as jnp
from jax import lax
from jax.experimental import pallas as pl
from jax.experimental.pallas import tpu as pltpu
from jax.experimental.pallas import tpu_sc as plsc

N_POINTS = 524288
N_TREES = 16
LEVEL_RES = (32, 64, 128, 256, 512, 1024)
LOG2_HASHMAP = 17
LEVEL_SIZES = tuple(int(min(r**3, 2**LOG2_HASHMAP)) for r in LEVEL_RES)
ROW_OFFS = tuple(sum(LEVEL_SIZES[:i]) for i in range(len(LEVEL_SIZES)))
TOTAL_ROWS = sum(LEVEL_SIZES)
N_LEVELS = len(LEVEL_RES)
N_OUT = 2 * N_LEVELS
N_WORDS = N_TREES * TOTAL_ROWS * 2

P1 = int(2654435761 - 2**32)
P2 = 805459861

NC, NS = 2, 16
NW = NC * NS
PPT = N_POINTS // NW
BT = 128
NCHUNK = PPT // BT
NGRP = BT // 16
IDXW = 8 * N_LEVELS * 2 * BT
W = 2048
NDMA = IDXW // W


def _encode_sc(xf, inds, table):
    mesh = plsc.VectorSubcoreMesh(core_axis_name="c", subcore_axis_name="s")

    @functools.partial(
        pl.kernel,
        out_type=jax.ShapeDtypeStruct((N_POINTS, N_OUT), jnp.float32),
        mesh=mesh,
        compiler_params=pltpu.CompilerParams(
            needs_layout_passes=False, use_tc_tiling_on_sc=False),
        scratch_types=[
            pltpu.VMEM((2, 3 * BT), jnp.float32),
            pltpu.VMEM((2, BT), jnp.int32),
            pltpu.VMEM((2, NDMA, W), jnp.int32),
            pltpu.VMEM((2, NDMA, W), jnp.float32),
            pltpu.VMEM((2, BT, N_OUT), jnp.float32),
            pltpu.SemaphoreType.DMA((2,)),
            pltpu.SemaphoreType.DMA((2,)),
            pltpu.SemaphoreType.DMA((2,)),
        ],
    )
    def body(xflat_hbm, inds_hbm, table_hbm, out_hbm, xf_v, ind_v, idx_v,
             rows_v, out_v, gsem, isem, osem):
        wid = lax.axis_index("s") * NC + lax.axis_index("c")
        lane = lax.iota(jnp.int32, 16)
        lane3 = lane * 3
        tbase = wid * PPT

        def start_in(i, b):
            cbase = tbase + i * BT
            pltpu.async_copy(xflat_hbm.at[pl.ds(cbase * 3, BT * 3)],
                             xf_v.at[b], isem.at[b])
            pltpu.async_copy(inds_hbm.at[pl.ds(cbase, BT)], ind_v.at[b],
                             isem.at[b])

        def wait_in(i, b):
            cbase = tbase + i * BT
            pltpu.make_async_copy(xflat_hbm.at[pl.ds(cbase * 3, BT * 3)],
                                  xf_v.at[b], isem.at[b]).wait()
            pltpu.make_async_copy(inds_hbm.at[pl.ds(cbase, BT)],
                                  ind_v.at[b], isem.at[b]).wait()

        def phase_a(b):
            def grp_idx(g, c2):
                p = g * 16
                bvec = lane * 0 + b
                cx = p * 3 + lane3
                ux = plsc.load_gather(xf_v, [bvec, cx]) * 0.5 + 0.5
                uy = plsc.load_gather(xf_v, [bvec, cx + 1]) * 0.5 + 0.5
                uz = plsc.load_gather(xf_v, [bvec, cx + 2]) * 0.5 + 0.5
                tw = ind_v[b, pl.ds(p, 16)] * (TOTAL_ROWS * 2)
                for l in range(N_LEVELS):
                    r = LEVEL_RES[l]
                    mask = LEVEL_SIZES[l] - 1
                    px = jnp.clip((ux * (r - 1)).astype(jnp.int32), 0, r - 2)
                    py = jnp.clip((uy * (r - 1)).astype(jnp.int32), 0, r - 2)
                    pz = jnp.clip((uz * (r - 1)).astype(jnp.int32), 0, r - 2)
                    hy0 = py * P1
                    hz0 = pz * P2
                    hx = (px, px + 1)
                    hy = (hy0, hy0 + P1)
                    hz = (hz0, hz0 + P2)
                    base_l = tw + 2 * ROW_OFFS[l]
                    for c in range(8):
                        h = (hx[c & 1] ^ hy[(c >> 1) & 1] ^ hz[(c >> 2) & 1])
                        w0 = base_l + ((h & mask) << 1)
                        q = (l * 8 + c) * (2 * BT)
                        idx_v[b, q // W, pl.ds(q % W + p, 16)] = w0
                        q1 = q + BT
                        idx_v[b, q1 // W, pl.ds(q1 % W + p, 16)] = w0 + 1
                return c2

            lax.fori_loop(0, NGRP, grp_idx, 0)

            def fire(j, c2):
                pltpu.async_copy(table_hbm.at[idx_v.at[b, j]],
                                 rows_v.at[b, j], gsem.at[b])
                return c2

            lax.fori_loop(0, NDMA, fire, 0)

        def drain_gather(b):
            def drain(j, c2):
                pltpu.make_async_copy(table_hbm.at[idx_v.at[b, j]],
                                      rows_v.at[b, j], gsem.at[b]).wait()
                return c2

            lax.fori_loop(0, NDMA, drain, 0)

        def phase_b(i, b):
            @pl.when(i >= 2)
            def _():
                cb2 = tbase + (i - 2) * BT
                pltpu.make_async_copy(out_v.at[b],
                                      out_hbm.at[pl.ds(cb2, BT)],
                                      osem.at[b]).wait()

            def grp_acc(g, c2):
                p = g * 16
                ridx = p + lane
                bvec = lane * 0 + b
                cx = p * 3 + lane3
                ux = plsc.load_gather(xf_v, [bvec, cx]) * 0.5 + 0.5
                uy = plsc.load_gather(xf_v, [bvec, cx + 1]) * 0.5 + 0.5
                uz = plsc.load_gather(xf_v, [bvec, cx + 2]) * 0.5 + 0.5
                for l in range(N_LEVELS):
                    r = LEVEL_RES[l]
                    posx = ux * (r - 1)
                    posy = uy * (r - 1)
                    posz = uz * (r - 1)
                    px = jnp.clip(posx.astype(jnp.int32), 0, r - 2)
                    py = jnp.clip(posy.astype(jnp.int32), 0, r - 2)
                    pz = jnp.clip(posz.astype(jnp.int32), 0, r - 2)
                    fx = posx - px.astype(jnp.float32)
                    fy = posy - py.astype(jnp.float32)
                    fz = posz - pz.astype(jnp.float32)
                    wx = (1.0 - fx, fx)
                    wy = (1.0 - fy, fy)
                    wz = (1.0 - fz, fz)
                    acc0 = jnp.zeros((16,), jnp.float32)
                    acc1 = jnp.zeros((16,), jnp.float32)
                    for c in range(8):
                        q = (l * 8 + c) * (2 * BT)
                        q1 = q + BT
                        f0 = rows_v[b, q // W, pl.ds(q % W + p, 16)]
                        f1 = rows_v[b, q1 // W, pl.ds(q1 % W + p, 16)]
                        w = wx[c & 1] * wy[(c >> 1) & 1] * wz[(c >> 2) & 1]
                        acc0 = acc0 + w * f0
                        acc1 = acc1 + w * f1
                    plsc.store_scatter(out_v, [lane * 0 + b, ridx,
                                               lane * 0 + 2 * l], acc0)
                    plsc.store_scatter(out_v, [lane * 0 + b, ridx,
                                               lane * 0 + 2 * l + 1], acc1)
                return c2

            lax.fori_loop(0, NGRP, grp_acc, 0)
            cbase = tbase + i * BT
            pltpu.async_copy(out_v.at[b], out_hbm.at[pl.ds(cbase, BT)],
                             osem.at[b])

        start_in(0, 0)
        wait_in(0, 0)
        phase_a(0)
        start_in(1, 1)

        def pipe(i, carry):
            b = i & 1
            nb = 1 - b

            @pl.when(i + 1 < NCHUNK)
            def _():
                wait_in(i + 1, nb)
                phase_a(nb)

            drain_gather(b)
            phase_b(i, b)

            @pl.when(i + 2 < NCHUNK)
            def _():
                start_in(i + 2, b)
            return carry

        lax.fori_loop(0, NCHUNK, pipe, 0)
        for i in (NCHUNK - 2, NCHUNK - 1):
            cb = tbase + i * BT
            pltpu.make_async_copy(out_v.at[i & 1],
                                  out_hbm.at[pl.ds(cb, BT)],
                                  osem.at[i & 1]).wait()

    return body(xf, inds, table)


def kernel(block_x, block_inds, forest_flattened_params):
    return _encode_sc(block_x.reshape(N_POINTS * 3),
                      block_inds.astype(jnp.int32),
                      forest_flattened_params.reshape(N_WORDS))


# speedup vs baseline: 1.3153x; 1.3153x over previous
import functools

import jax
import jax.numpy as jnp
from jax import lax
from jax.experimental import pallas as pl
from jax.experimental.pallas import tpu as pltpu
from jax.experimental.pallas import tpu_sc as plsc

N_POINTS = 524288
N_TREES = 16
LEVEL_RES = (32, 64, 128, 256, 512, 1024)
LOG2_HASHMAP = 17
LEVEL_SIZES = tuple(int(min(r**3, 2**LOG2_HASHMAP)) for r in LEVEL_RES)
ROW_OFFS = tuple(sum(LEVEL_SIZES[:i]) for i in range(len(LEVEL_SIZES)))
TOTAL_ROWS = sum(LEVEL_SIZES)
N_LEVELS = len(LEVEL_RES)
N_OUT = 2 * N_LEVELS
N_WORDS = N_TREES * TOTAL_ROWS * 2

P1 = int(2654435761 - 2**32)
P2 = 805459861

NC, NS = 2, 16
NW = NC * NS
PPT = N_POINTS // NW
BT = 128
NCHUNK = PPT // BT
NGRP = BT // 16
IDXW = 8 * N_LEVELS * 2 * BT
W = 2048
NDMA = IDXW // W


def _encode_sc(xt, inds, table):
    mesh = plsc.VectorSubcoreMesh(core_axis_name="c", subcore_axis_name="s")

    @functools.partial(
        pl.kernel,
        out_type=jax.ShapeDtypeStruct((N_POINTS, N_OUT), jnp.float32),
        mesh=mesh,
        compiler_params=pltpu.CompilerParams(
            needs_layout_passes=False, use_tc_tiling_on_sc=False),
        scratch_types=[
            pltpu.VMEM((2, 3, BT), jnp.float32),
            pltpu.VMEM((2, BT), jnp.int32),
            pltpu.VMEM((2, NDMA, W), jnp.int32),
            pltpu.VMEM((2, NDMA, W), jnp.float32),
            pltpu.VMEM((2, BT, N_OUT), jnp.float32),
            pltpu.SemaphoreType.DMA((2,)),
            pltpu.SemaphoreType.DMA((2,)),
            pltpu.SemaphoreType.DMA((2,)),
        ],
    )
    def body(xt_hbm, inds_hbm, table_hbm, out_hbm, xyz_v, ind_v, idx_v,
             rows_v, out_v, gsem, isem, osem):
        wid = lax.axis_index("s") * NC + lax.axis_index("c")
        lane = lax.iota(jnp.int32, 16)
        tbase = wid * PPT

        def start_in(i, b):
            cbase = tbase + i * BT
            pltpu.async_copy(xt_hbm.at[:, pl.ds(cbase, BT)], xyz_v.at[b],
                             isem.at[b])
            pltpu.async_copy(inds_hbm.at[pl.ds(cbase, BT)], ind_v.at[b],
                             isem.at[b])

        def wait_in(i, b):
            cbase = tbase + i * BT
            pltpu.make_async_copy(xt_hbm.at[:, pl.ds(cbase, BT)],
                                  xyz_v.at[b], isem.at[b]).wait()
            pltpu.make_async_copy(inds_hbm.at[pl.ds(cbase, BT)],
                                  ind_v.at[b], isem.at[b]).wait()

        def phase_a(b):
            def grp_idx(g, c2):
                p = g * 16
                ux = xyz_v[b, 0, pl.ds(p, 16)] * 0.5 + 0.5
                uy = xyz_v[b, 1, pl.ds(p, 16)] * 0.5 + 0.5
                uz = xyz_v[b, 2, pl.ds(p, 16)] * 0.5 + 0.5
                tw = ind_v[b, pl.ds(p, 16)] * (TOTAL_ROWS * 2)
                for l in range(N_LEVELS):
                    r = LEVEL_RES[l]
                    mask = LEVEL_SIZES[l] - 1
                    px = jnp.clip((ux * (r - 1)).astype(jnp.int32), 0, r - 2)
                    py = jnp.clip((uy * (r - 1)).astype(jnp.int32), 0, r - 2)
                    pz = jnp.clip((uz * (r - 1)).astype(jnp.int32), 0, r - 2)
                    hy0 = py * P1
                    hz0 = pz * P2
                    hx = (px, px + 1)
                    hy = (hy0, hy0 + P1)
                    hz = (hz0, hz0 + P2)
                    base_l = tw + 2 * ROW_OFFS[l]
                    for c in range(8):
                        h = (hx[c & 1] ^ hy[(c >> 1) & 1] ^ hz[(c >> 2) & 1])
                        w0 = base_l + ((h & mask) << 1)
                        q = (l * 8 + c) * (2 * BT)
                        idx_v[b, q // W, pl.ds(q % W + p, 16)] = w0
                        q1 = q + BT
                        idx_v[b, q1 // W, pl.ds(q1 % W + p, 16)] = w0 + 1
                return c2

            lax.fori_loop(0, NGRP, grp_idx, 0)

            def fire(j, c2):
                pltpu.async_copy(table_hbm.at[idx_v.at[b, j]],
                                 rows_v.at[b, j], gsem.at[b])
                return c2

            lax.fori_loop(0, NDMA, fire, 0)

        def drain_gather(b):
            def drain(j, c2):
                pltpu.make_async_copy(table_hbm.at[idx_v.at[b, j]],
                                      rows_v.at[b, j], gsem.at[b]).wait()
                return c2

            lax.fori_loop(0, NDMA, drain, 0)

        def phase_b(i, b):
            @pl.when(i >= 2)
            def _():
                cb2 = tbase + (i - 2) * BT
                pltpu.make_async_copy(out_v.at[b],
                                      out_hbm.at[pl.ds(cb2, BT)],
                                      osem.at[b]).wait()

            def grp_acc(g, c2):
                p = g * 16
                ridx = p + lane
                ux = xyz_v[b, 0, pl.ds(p, 16)] * 0.5 + 0.5
                uy = xyz_v[b, 1, pl.ds(p, 16)] * 0.5 + 0.5
                uz = xyz_v[b, 2, pl.ds(p, 16)] * 0.5 + 0.5
                for l in range(N_LEVELS):
                    r = LEVEL_RES[l]
                    posx = ux * (r - 1)
                    posy = uy * (r - 1)
                    posz = uz * (r - 1)
                    px = jnp.clip(posx.astype(jnp.int32), 0, r - 2)
                    py = jnp.clip(posy.astype(jnp.int32), 0, r - 2)
                    pz = jnp.clip(posz.astype(jnp.int32), 0, r - 2)
                    fx = posx - px.astype(jnp.float32)
                    fy = posy - py.astype(jnp.float32)
                    fz = posz - pz.astype(jnp.float32)
                    wx = (1.0 - fx, fx)
                    wy = (1.0 - fy, fy)
                    wz = (1.0 - fz, fz)
                    acc0 = jnp.zeros((16,), jnp.float32)
                    acc1 = jnp.zeros((16,), jnp.float32)
                    for c in range(8):
                        q = (l * 8 + c) * (2 * BT)
                        q1 = q + BT
                        f0 = rows_v[b, q // W, pl.ds(q % W + p, 16)]
                        f1 = rows_v[b, q1 // W, pl.ds(q1 % W + p, 16)]
                        w = wx[c & 1] * wy[(c >> 1) & 1] * wz[(c >> 2) & 1]
                        acc0 = acc0 + w * f0
                        acc1 = acc1 + w * f1
                    plsc.store_scatter(out_v, [lane * 0 + b, ridx,
                                               lane * 0 + 2 * l], acc0)
                    plsc.store_scatter(out_v, [lane * 0 + b, ridx,
                                               lane * 0 + 2 * l + 1], acc1)
                return c2

            lax.fori_loop(0, NGRP, grp_acc, 0)
            cbase = tbase + i * BT
            pltpu.async_copy(out_v.at[b], out_hbm.at[pl.ds(cbase, BT)],
                             osem.at[b])

        start_in(0, 0)
        wait_in(0, 0)
        phase_a(0)
        start_in(1, 1)

        def pipe(i, carry):
            b = i & 1
            nb = 1 - b

            @pl.when(i + 1 < NCHUNK)
            def _():
                wait_in(i + 1, nb)
                phase_a(nb)

            drain_gather(b)
            phase_b(i, b)

            @pl.when(i + 2 < NCHUNK)
            def _():
                start_in(i + 2, b)
            return carry

        lax.fori_loop(0, NCHUNK, pipe, 0)
        for i in (NCHUNK - 2, NCHUNK - 1):
            cb = tbase + i * BT
            pltpu.make_async_copy(out_v.at[i & 1],
                                  out_hbm.at[pl.ds(cb, BT)],
                                  osem.at[i & 1]).wait()

    return body(xt, inds, table)


def kernel(block_x, block_inds, forest_flattened_params):
    table = forest_flattened_params.reshape(N_WORDS)
    return _encode_sc(block_x.T, block_inds.astype(jnp.int32), table)
